# baseline (speedup 1.0000x reference)
import functools

import jax
import jax.numpy as jnp
from jax.experimental import pallas as pl
from jax.experimental.pallas import tpu as pltpu

_N = 5000
_B = 512
_NP = 5120
_NB = _NP // _B
_NMS_T = 0.5
_SCORE_T = 0.05
_TOPK = 100


def _nms_body(rows_ref, cols_ref, keep_ref, mrows_ref):
    ri = jax.lax.broadcasted_iota(jnp.int32, (_B, _B), 0)
    ci = jax.lax.broadcasted_iota(jnp.int32, (_B, _B), 1)
    upper = (ci < ri).astype(jnp.float32)
    eye = (ri == ci)

    def outer(i, _):
        kb = i * _B
        kx1 = cols_ref[pl.ds(kb, _B), 0:1]
        ky1 = cols_ref[pl.ds(kb, _B), 1:2]
        kx2 = cols_ref[pl.ds(kb, _B), 2:3]
        ky2 = cols_ref[pl.ds(kb, _B), 3:4]
        kval = cols_ref[pl.ds(kb, _B), 4:5]
        karea = cols_ref[pl.ds(kb, _B), 5:6]

        def iou_vs_rows(src, jb):
            jx1 = src[0:1, pl.ds(jb, _B)]
            jy1 = src[1:2, pl.ds(jb, _B)]
            jx2 = src[2:3, pl.ds(jb, _B)]
            jy2 = src[3:4, pl.ds(jb, _B)]
            jarea = src[4:5, pl.ds(jb, _B)]
            w = jnp.maximum(jnp.minimum(kx2, jx2) - jnp.maximum(kx1, jx1), 0.0)
            h = jnp.maximum(jnp.minimum(ky2, jy2) - jnp.maximum(ky1, jy1), 0.0)
            inter = w * h
            return inter / (karea + jarea - inter + 1e-9)

        def cross(j, acc):
            iou = iou_vs_rows(mrows_ref, j * _B)
            return jnp.maximum(acc, jnp.max(iou, axis=1, keepdims=True))

        mx = jax.lax.fori_loop(0, i, cross, jnp.zeros((_B, 1), jnp.float32))
        v = kval * jnp.where(mx > _NMS_T, 0.0, 1.0)

        iou_ii = iou_vs_rows(rows_ref, kb)
        m = jnp.where(iou_ii > _NMS_T, 1.0, 0.0) * upper

        def cond(carry):
            return carry[1]

        def body(carry):
            a, _ = carry
            s = jax.lax.dot_general(
                m, a, (((1,), (0,)), ((), ())),
                preferred_element_type=jnp.float32)
            anew = jnp.where(s > 0.5, 0.0, v)
            return anew, jnp.any(anew != a)

        a, _ = jax.lax.while_loop(cond, body, (v, jnp.bool_(True)))

        a_row = jnp.sum(jnp.where(eye, a, 0.0), axis=0, keepdims=True)
        keep_ref[0:1, pl.ds(kb, _B)] = a_row
        mrows_ref[:, pl.ds(kb, _B)] = rows_ref[:, pl.ds(kb, _B)] * a_row
        return 0

    jax.lax.fori_loop(0, _NB, outer, 0)


@functools.partial(jax.jit, static_argnames=())
def kernel(boxes, scores):
    iota = jnp.arange(_N, dtype=jnp.int32)
    neg, _, x1, y1, x2, y2 = jax.lax.sort(
        (-scores, iota, boxes[:, 0], boxes[:, 1], boxes[:, 2], boxes[:, 3]),
        dimension=0, num_keys=2, is_stable=False)
    s = -neg
    b = jnp.stack([x1, y1, x2, y2], axis=1)

    valid = (s > _SCORE_T).astype(jnp.float32)
    area = (x2 - x1) * (y2 - y1)
    rows = (jnp.zeros((8, _NP), jnp.float32)
            .at[0:4, 0:_N].set(b.T)
            .at[4, 0:_N].set(area))
    cols = (jnp.zeros((_NP, 8), jnp.float32)
            .at[0:_N, 0:4].set(b)
            .at[0:_N, 4].set(valid)
            .at[0:_N, 5].set(area))

    keep = pl.pallas_call(
        _nms_body,
        out_shape=jax.ShapeDtypeStruct((1, _NP), jnp.float32),
        scratch_shapes=[pltpu.VMEM((8, _NP), jnp.float32)],
    )(rows, cols)

    keep_n = keep[0, 0:_N] > 0.5
    r = jnp.arange(_TOPK, dtype=jnp.int32)
    ck = jnp.cumsum(keep_n.astype(jnp.int32))
    n_keep = ck[_N - 1]
    kept_idx = jnp.searchsorted(ck, r + 1, side="left", method="compare_all")
    cn = jnp.cumsum((~keep_n).astype(jnp.int32))
    tail_idx = jnp.searchsorted(cn, r - n_keep + 1, side="left",
                                method="compare_all")
    is_kept_slot = r < n_keep
    idx = jnp.clip(jnp.where(is_kept_slot, kept_idx, tail_idx), 0, _N - 1)
    top_scores = jnp.where(is_kept_slot, s[idx], -jnp.inf)
    top_boxes = b[idx]
    return jnp.concatenate([top_boxes, top_scores[:, None]], axis=1)


# speedup vs baseline: 1.0272x; 1.0272x over previous
import functools

import jax
import jax.numpy as jnp
from jax.experimental import pallas as pl
from jax.experimental.pallas import tpu as pltpu

_N = 5000
_B = 1024
_NP = 5120
_NB = _NP // _B
_NMS_T = 0.5
_SCORE_T = 0.05
_TOPK = 100


def _nms_body(rows_ref, cols_ref, keep_ref, mrows_ref):
    ri = jax.lax.broadcasted_iota(jnp.int32, (_B, _B), 0)
    ci = jax.lax.broadcasted_iota(jnp.int32, (_B, _B), 1)
    upper = (ci < ri).astype(jnp.float32)
    eye = (ri == ci)

    def outer(i, _):
        kb = i * _B
        kx1 = cols_ref[pl.ds(kb, _B), 0:1]
        ky1 = cols_ref[pl.ds(kb, _B), 1:2]
        kx2 = cols_ref[pl.ds(kb, _B), 2:3]
        ky2 = cols_ref[pl.ds(kb, _B), 3:4]
        kval = cols_ref[pl.ds(kb, _B), 4:5]
        karea = cols_ref[pl.ds(kb, _B), 5:6]

        def iou_vs_rows(src, jb):
            jx1 = src[0:1, pl.ds(jb, _B)]
            jy1 = src[1:2, pl.ds(jb, _B)]
            jx2 = src[2:3, pl.ds(jb, _B)]
            jy2 = src[3:4, pl.ds(jb, _B)]
            jarea = src[4:5, pl.ds(jb, _B)]
            w = jnp.maximum(jnp.minimum(kx2, jx2) - jnp.maximum(kx1, jx1), 0.0)
            h = jnp.maximum(jnp.minimum(ky2, jy2) - jnp.maximum(ky1, jy1), 0.0)
            inter = w * h
            return inter / (karea + jarea - inter + 1e-9)

        def cross(j, acc):
            iou = iou_vs_rows(mrows_ref, j * _B)
            return jnp.maximum(acc, jnp.max(iou, axis=1, keepdims=True))

        mx = jax.lax.fori_loop(0, i, cross, jnp.zeros((_B, 1), jnp.float32))
        v = kval * jnp.where(mx > _NMS_T, 0.0, 1.0)

        iou_ii = iou_vs_rows(rows_ref, kb)
        m = jnp.where(iou_ii > _NMS_T, 1.0, 0.0) * upper

        def cond(carry):
            return carry[1]

        def body(carry):
            a, _ = carry
            s = jax.lax.dot_general(
                m, a, (((1,), (0,)), ((), ())),
                preferred_element_type=jnp.float32)
            anew = jnp.where(s > 0.5, 0.0, v)
            return anew, jnp.any(anew != a)

        a, _ = jax.lax.while_loop(cond, body, (v, jnp.bool_(True)))

        a_row = jnp.sum(jnp.where(eye, a, 0.0), axis=0, keepdims=True)
        keep_ref[0:1, pl.ds(kb, _B)] = a_row
        mrows_ref[:, pl.ds(kb, _B)] = rows_ref[:, pl.ds(kb, _B)] * a_row
        return 0

    jax.lax.fori_loop(0, _NB, outer, 0)


@functools.partial(jax.jit, static_argnames=())
def kernel(boxes, scores):
    iota = jnp.arange(_N, dtype=jnp.int32)
    neg, _, x1, y1, x2, y2 = jax.lax.sort(
        (-scores, iota, boxes[:, 0], boxes[:, 1], boxes[:, 2], boxes[:, 3]),
        dimension=0, num_keys=2, is_stable=False)
    s = -neg
    b = jnp.stack([x1, y1, x2, y2], axis=1)

    valid = (s > _SCORE_T).astype(jnp.float32)
    area = (x2 - x1) * (y2 - y1)
    rows = (jnp.zeros((8, _NP), jnp.float32)
            .at[0:4, 0:_N].set(b.T)
            .at[4, 0:_N].set(area))
    cols = (jnp.zeros((_NP, 8), jnp.float32)
            .at[0:_N, 0:4].set(b)
            .at[0:_N, 4].set(valid)
            .at[0:_N, 5].set(area))

    keep = pl.pallas_call(
        _nms_body,
        out_shape=jax.ShapeDtypeStruct((1, _NP), jnp.float32),
        scratch_shapes=[pltpu.VMEM((8, _NP), jnp.float32)],
    )(rows, cols)

    keep_n = keep[0, 0:_N] > 0.5
    r = jnp.arange(_TOPK, dtype=jnp.int32)
    ck = jnp.cumsum(keep_n.astype(jnp.int32))
    n_keep = ck[_N - 1]
    kept_idx = jnp.searchsorted(ck, r + 1, side="left", method="compare_all")
    cn = jnp.cumsum((~keep_n).astype(jnp.int32))
    tail_idx = jnp.searchsorted(cn, r - n_keep + 1, side="left",
                                method="compare_all")
    is_kept_slot = r < n_keep
    idx = jnp.clip(jnp.where(is_kept_slot, kept_idx, tail_idx), 0, _N - 1)
    top_scores = jnp.where(is_kept_slot, s[idx], -jnp.inf)
    top_boxes = b[idx]
    return jnp.concatenate([top_boxes, top_scores[:, None]], axis=1)
